# baseline (speedup 1.0000x reference)
import functools

import jax
import jax.numpy as jnp
from jax import lax
from jax.experimental import pallas as pl
from jax.experimental.pallas import tpu as pltpu
from jax.experimental.pallas import tpu_sc as plsc

_NG = 512
_CH = 2048
_NVEC = _CH // 16
_CAPB = _CH // 128 + 1

_SC_PARAMS = pltpu.CompilerParams(needs_layout_passes=False,
                                  use_tc_tiling_on_sc=False)


def _agg_body(x_hbm, src_hbm, dst_hbm, zeros_hbm, out_hbm,
              acc, stage_src, stage_dst, srcbuf, dstbuf, rows, sem,
              *, base0, R, n_chunks):
    c = lax.axis_index("c")
    s = lax.axis_index("s")
    et0 = s * (n_chunks * _CH)
    zs = (R + 128) // 16
    ws = R // 16
    base = base0 + c * R

    lane = lax.iota(jnp.int32, 16)
    zero16 = jnp.zeros((16,), jnp.int32)
    dummy16 = jnp.full((16,), R, jnp.int32)

    def filter_vec(i, cnt):
        dv = stage_dst[pl.ds(i * 16, 16)]
        sv = stage_src[pl.ds(i * 16, 16)]
        loc = dv - base
        t = jnp.bitwise_or(loc, (R - 1) - loc)
        neg = jnp.right_shift(t, 31)
        v1 = 1 + neg
        key = lane - neg * 4096
        packed = jnp.left_shift(sv * v1, 14) + loc * v1 - neg * R
        _, sorted_v = plsc.sort_key_val(key, packed)
        srcv = jnp.right_shift(sorted_v, 14)
        locv = jnp.bitwise_and(sorted_v, 16383)
        pos = cnt + lane
        plsc.store_scatter(srcbuf, [pos], srcv)
        plsc.store_scatter(dstbuf,
                           [jnp.right_shift(pos, 7),
                            jnp.bitwise_and(pos, 127)], locv)
        return cnt + 16 + jnp.sum(neg)

    def gather_block(j, _):
        pltpu.async_copy(x_hbm.at[srcbuf.at[pl.ds(j * 128, 128)]], rows,
                         sem).wait()
        pltpu.sync_copy(rows, acc.at[dstbuf.at[j]], add=True)
        return 0

    def chunk_body(k, _):
        e0 = et0 + k * _CH
        pltpu.sync_copy(src_hbm.at[pl.ds(e0, _CH)], stage_src)
        pltpu.sync_copy(dst_hbm.at[pl.ds(e0, _CH)], stage_dst)
        cnt = lax.fori_loop(0, _NVEC, filter_vec, jnp.int32(0))
        for q in range(8):
            pos = cnt + lane + q * 16
            plsc.store_scatter(srcbuf, [pos], zero16)
            plsc.store_scatter(dstbuf,
                               [jnp.right_shift(pos, 7),
                                jnp.bitwise_and(pos, 127)], dummy16)
        nblk = jnp.right_shift(cnt + 127, 7)
        lax.fori_loop(0, nblk, gather_block, 0)
        return 0

    pltpu.sync_copy(zeros_hbm.at[pl.ds(0, zs)], acc.at[pl.ds(s * zs, zs)])
    plsc.subcore_barrier()
    lax.fori_loop(0, n_chunks, chunk_body, 0)
    plsc.subcore_barrier()
    pltpu.sync_copy(acc.at[pl.ds(s * ws, ws)],
                    out_hbm.at[pl.ds(c * R + s * ws, ws)])


def _sc_agg1(xp, src, dst, zeros_hbm, *, base0, R, n_chunks):
    dp = xp.shape[1]
    mesh = plsc.VectorSubcoreMesh(core_axis_name="c", subcore_axis_name="s")
    body = functools.partial(_agg_body, base0=base0, R=R, n_chunks=n_chunks)
    k = pl.kernel(
        body,
        mesh=mesh,
        compiler_params=_SC_PARAMS,
        out_type=jax.ShapeDtypeStruct((2 * R, dp), jnp.float32),
        scratch_types=[
            pltpu.VMEM_SHARED((R + 128, dp), jnp.float32),
            pltpu.VMEM((_CH,), jnp.int32),
            pltpu.VMEM((_CH,), jnp.int32),
            pltpu.VMEM((_CAPB * 128,), jnp.int32),
            pltpu.VMEM((_CAPB, 128), jnp.int32),
            pltpu.VMEM((128, dp), jnp.float32),
            pltpu.SemaphoreType.DMA,
        ],
    )
    return k(xp, src, dst, zeros_hbm)


def _sc_agg(xp, src, dst, zeros_hbm, *, P, R, n_chunks):
    np_rows = xp.shape[0]
    parts = [_sc_agg1(xp, src, dst, zeros_hbm, base0=2 * p * R, R=R,
                      n_chunks=n_chunks) for p in range(P)]
    agg = jnp.concatenate(parts, axis=0)
    return jnp.pad(agg, ((0, np_rows - 2 * P * R), (0, 0)))


def _pool_body(x_hbm, batch_hbm, zeros_hbm, out_hbm,
               acc, stage, rows, *, J, B):
    c = lax.axis_index("c")
    s = lax.axis_index("s")
    wid = s * 2 + c
    r0 = wid * (J * B)
    pltpu.sync_copy(zeros_hbm.at[pl.ds(0, 40)], acc.at[pl.ds(s * 40, 40)])
    pltpu.sync_copy(batch_hbm.at[wid], stage)
    plsc.subcore_barrier()

    def blk(j, _):
        pltpu.sync_copy(x_hbm.at[pl.ds(r0 + j * B, B)], rows)
        pltpu.sync_copy(rows, acc.at[stage.at[j]], add=True)
        return 0

    lax.fori_loop(0, J, blk, 0)
    plsc.subcore_barrier()
    pltpu.sync_copy(acc.at[pl.ds(s * 32, 32)],
                    out_hbm.at[c, pl.ds(s * 32, 32)])


def _sc_pool(xp, batch2d, zeros_hbm):
    J, B = batch2d.shape[1], batch2d.shape[2]
    mesh = plsc.VectorSubcoreMesh(core_axis_name="c", subcore_axis_name="s")
    body = functools.partial(_pool_body, J=J, B=B)
    k = pl.kernel(
        body,
        mesh=mesh,
        compiler_params=_SC_PARAMS,
        out_type=jax.ShapeDtypeStruct((2, _NG, 144), jnp.float32),
        scratch_types=[
            pltpu.VMEM_SHARED((640, 144), jnp.float32),
            pltpu.VMEM((J, B), jnp.int32),
            pltpu.VMEM((B, 144), jnp.float32),
        ],
    )
    return k(xp, batch2d, zeros_hbm)


def _dense_body(a_ref, x_ref, wl_ref, wr_ref, b_ref, o_ref):
    acc = jnp.dot(a_ref[...], wl_ref[...],
                  preferred_element_type=jnp.float32,
                  precision=lax.Precision.HIGHEST)
    acc = acc + jnp.dot(x_ref[...], wr_ref[...],
                        preferred_element_type=jnp.float32,
                        precision=lax.Precision.HIGHEST)
    o_ref[...] = jnp.maximum(acc + b_ref[...], 0.0)


def _dense(aggp, xp, wlt, wrt, bp):
    np_rows, dpi = xp.shape
    dpo = wlt.shape[1]
    br = 512
    return pl.pallas_call(
        _dense_body,
        grid=(np_rows // br,),
        in_specs=[
            pl.BlockSpec((br, dpi), lambda i: (i, 0)),
            pl.BlockSpec((br, dpi), lambda i: (i, 0)),
            pl.BlockSpec((dpi, dpo), lambda i: (0, 0)),
            pl.BlockSpec((dpi, dpo), lambda i: (0, 0)),
            pl.BlockSpec((1, dpo), lambda i: (0, 0)),
        ],
        out_specs=pl.BlockSpec((br, dpo), lambda i: (i, 0)),
        out_shape=jax.ShapeDtypeStruct((np_rows, dpo), jnp.float32),
    )(aggp, xp, wlt, wrt, bp)


def _combine_body(mp_ref, cp_ref, o_ref):
    ms = mp_ref[0] + mp_ref[1]
    cs = cp_ref[0] + cp_ref[1]
    o_ref[:, :128] = ms[:, :128] / jnp.maximum(ms[:, 128:129], 1.0)
    o_ref[:, 128:] = cs[:, :128] / jnp.maximum(cs[:, 128:129], 1.0)


def _padw(W, dpi, dpo):
    return jnp.zeros((dpi, dpo), jnp.float32).at[:W.shape[1], :W.shape[0]].set(W.T)


def _padb(b, dpo, ones_col=False):
    bp = jnp.zeros((1, dpo), jnp.float32).at[0, :b.shape[0]].set(b)
    if ones_col:
        bp = bp.at[0, 128].set(1.0)
    return bp


def _branch(x, edge_index, batch, n, np_rows, e_pad, dims, passes, ranges,
            pool_jb, weights):
    (W1l, b1, W1r, W2l, b2, W2r, W3l, b3, W3r) = weights
    dp0, dp1, dp2 = dims
    d0 = x.shape[1]
    n_edges = edge_index.shape[1]
    n_chunks = e_pad // (16 * _CH)

    src = jnp.zeros((e_pad,), jnp.int32).at[:n_edges].set(
        edge_index[0].astype(jnp.int32))
    dst = jnp.full((e_pad,), -1, jnp.int32).at[:n_edges].set(
        edge_index[1].astype(jnp.int32))

    zeros_hbm = jnp.zeros((800, 368), jnp.float32)

    xp = jnp.zeros((np_rows, dp0), jnp.float32).at[:n, :d0].set(x)

    a1 = _sc_agg(xp, src, dst, zeros_hbm[:, :dp0],
                 P=passes[0], R=ranges[0], n_chunks=n_chunks)
    h1 = _dense(a1, xp, _padw(W1l, dp0, dp1), _padw(W1r, dp0, dp1),
                _padb(b1, dp1))
    a2 = _sc_agg(h1, src, dst, zeros_hbm[:, :dp1],
                 P=passes[1], R=ranges[1], n_chunks=n_chunks)
    h2 = _dense(a2, h1, _padw(W2l, dp1, dp2), _padw(W2r, dp1, dp2),
                _padb(b2, dp2))
    a3 = _sc_agg(h2, src, dst, zeros_hbm[:, :dp2],
                 P=passes[2], R=ranges[2], n_chunks=n_chunks)
    h3 = _dense(a3, h2, _padw(W3l, dp2, 144), _padw(W3r, dp2, 144),
                _padb(b3, 144, ones_col=True))

    batch_pad = jnp.full((np_rows,), _NG, jnp.int32).at[:n].set(
        batch.astype(jnp.int32))
    batch2d = batch_pad.reshape(32, pool_jb[0], pool_jb[1])
    return _sc_pool(h3, batch2d, zeros_hbm[:, :144])


def kernel(mol_x, mol_edge_index, mol_batch, clique_x, clique_edge_index,
           clique_batch,
           mW1l, mb1, mW1r, mW2l, mb2, mW2r, mW3l, mb3, mW3r,
           cW1l, cb1, cW1r, cW2l, cb2, cW2r, cW3l, cb3, cW3r):
    mp = _branch(mol_x, mol_edge_index, mol_batch,
                 50000, 50688, 819200,
                 (80, 160, 320), (2, 4, 7), (12544, 6272, 3584),
                 (18, 88),
                 (mW1l, mb1, mW1r, mW2l, mb2, mW2r, mW3l, mb3, mW3r))
    cp = _branch(clique_x, clique_edge_index, clique_batch,
                 25000, 25600, 425984,
                 (96, 192, 368), (2, 3, 5), (6272, 4224, 2560),
                 (10, 80),
                 (cW1l, cb1, cW1r, cW2l, cb2, cW2r, cW3l, cb3, cW3r))
    return pl.pallas_call(
        _combine_body,
        out_shape=jax.ShapeDtypeStruct((_NG, 256), jnp.float32),
    )(mp, cp)
